# Initial kernel scaffold; baseline (speedup 1.0000x reference)
import jax
import jax.numpy as jnp
from jax.experimental import pallas as pl


def kernel(lv_coarse, ls_coarse, ls_fine, W, b):
    raise NotImplementedError("write your pallas kernel here")


# speedup vs baseline: 1.0782x; 1.0782x over previous
import functools

import jax
import jax.numpy as jnp
from jax import lax
from jax.experimental import pallas as pl
from jax.experimental.pallas import tpu as pltpu
from jax.experimental.pallas import tpu_sc as plsc

_LANES = 16
_CH = 128


def _project_tables(lv_coarse, w_blocks):
    n, d = lv_coarse.shape
    k_nbr, _, f = w_blocks.shape
    rb = 1000 if n % 1000 == 0 else 8
    assert n % rb == 0

    def body(lv_ref, w_ref, p_ref):
        p_ref[0] = jnp.dot(lv_ref[...], w_ref[0],
                           preferred_element_type=jnp.float32)

    return pl.pallas_call(
        body,
        grid=(n // rb, k_nbr),
        in_specs=[
            pl.BlockSpec((rb, d), lambda r, k: (r, 0)),
            pl.BlockSpec((1, d, f), lambda r, k: (k, 0, 0)),
        ],
        out_specs=pl.BlockSpec((1, rb, f), lambda r, k: (k, r, 0)),
        out_shape=jax.ShapeDtypeStruct((k_nbr, n, f), jnp.float32),
        compiler_params=pltpu.CompilerParams(
            dimension_semantics=("parallel", "arbitrary")),
    )(lv_coarse, w_blocks)


def _make_sc_gather_sum(k_nbr, f, bp, table_rows):
    info = plsc.get_sparse_core_info()
    nc, ns = info.num_cores, info.num_subcores
    nw = nc * ns
    b_per_w = bp // nw
    n_chunks = b_per_w // _CH
    jpf = f // _LANES

    mesh = plsc.VectorSubcoreMesh(core_axis_name="c", subcore_axis_name="s")

    @functools.partial(
        pl.kernel,
        mesh=mesh,
        out_type=jax.ShapeDtypeStruct((bp, f), jnp.float32),
        scratch_types=[
            pltpu.VMEM((k_nbr, _CH), jnp.int32),
            pltpu.VMEM((k_nbr, _CH, f), jnp.float32),
            pltpu.VMEM((f,), jnp.float32),
            pltpu.SemaphoreType.DMA,
        ],
    )
    def sc_kernel(table_hbm, idxt_hbm, b_hbm, out_hbm, idx_v, rows_v, bias_v,
                  sem):
        wid = lax.axis_index("s") * nc + lax.axis_index("c")
        base0 = wid * b_per_w
        pltpu.sync_copy(b_hbm, bias_v)
        bias_regs = [bias_v[pl.ds(_LANES * j, _LANES)] for j in range(jpf)]

        def chunk_body(c, carry):
            base = base0 + c * _CH
            for k in range(k_nbr):
                pltpu.sync_copy(idxt_hbm.at[k, pl.ds(base, _CH)],
                                idx_v.at[k])
            copies = [
                pltpu.async_copy(table_hbm.at[idx_v.at[k]], rows_v.at[k], sem)
                for k in range(k_nbr)
            ]
            for cp in copies:
                cp.wait()

            def row_body(i, carry2):
                for j in range(jpf):
                    sl = pl.ds(_LANES * j, _LANES)
                    acc = rows_v[0, i, sl]
                    for k in range(1, k_nbr):
                        acc = acc + rows_v[k, i, sl]
                    rows_v[0, i, sl] = jnp.maximum(acc + bias_regs[j], 0.0)
                return carry2

            lax.fori_loop(0, _CH, row_body, 0)
            pltpu.sync_copy(rows_v.at[0], out_hbm.at[pl.ds(base, _CH)])
            return carry

        lax.fori_loop(0, n_chunks, chunk_body, 0)

    return sc_kernel


def kernel(lv_coarse, ls_coarse, ls_fine, W, b):
    n_coarse, d = lv_coarse.shape
    n_fine, k_nbr = ls_fine.shape
    f = W.shape[1]

    w_blocks = W.reshape(k_nbr, d, f)
    p = _project_tables(lv_coarse, w_blocks)
    table = p.reshape(k_nbr * n_coarse, f)

    info = plsc.get_sparse_core_info()
    nw = info.num_cores * info.num_subcores
    gran = _CH * nw
    bp = ((n_fine + gran - 1) // gran) * gran

    idx = ls_fine.astype(jnp.int32)
    idx = idx + (jnp.arange(k_nbr, dtype=jnp.int32) * n_coarse)[None, :]
    idxt = jnp.pad(idx.T, ((0, 0), (0, bp - n_fine)))

    out = _make_sc_gather_sum(k_nbr, f, bp, k_nbr * n_coarse)(
        table, idxt, b)
    return out[:n_fine]


# speedup vs baseline: 2.3556x; 2.1848x over previous
import functools

import jax
import jax.numpy as jnp
from jax import lax
from jax.experimental import pallas as pl
from jax.experimental.pallas import tpu as pltpu
from jax.experimental.pallas import tpu_sc as plsc

_LANES = 16
_S = 32


def _project_tables(lv_coarse, w_blocks):
    n, d = lv_coarse.shape
    k_nbr, _, f = w_blocks.shape
    rb = 1000 if n % 1000 == 0 else 8
    assert n % rb == 0

    def body(lv_ref, w_ref, p_ref):
        p_ref[0] = jnp.dot(lv_ref[...], w_ref[0],
                           preferred_element_type=jnp.float32)

    return pl.pallas_call(
        body,
        grid=(n // rb, k_nbr),
        in_specs=[
            pl.BlockSpec((rb, d), lambda r, k: (r, 0)),
            pl.BlockSpec((1, d, f), lambda r, k: (k, 0, 0)),
        ],
        out_specs=pl.BlockSpec((1, rb, f), lambda r, k: (k, r, 0)),
        out_shape=jax.ShapeDtypeStruct((k_nbr, n, f), jnp.float32),
        compiler_params=pltpu.CompilerParams(
            dimension_semantics=("parallel", "arbitrary")),
    )(lv_coarse, w_blocks)


def _make_sc_gather_sum(k_nbr, f, n_fine):
    info = plsc.get_sparse_core_info()
    nc, ns = info.num_cores, info.num_subcores
    nw = nc * ns
    ips = _S * k_nbr
    n_sc_total = n_fine // _S
    assert n_fine % _S == 0
    per_w = -(-n_sc_total // nw)
    per_w += per_w % 2
    jpf = f // _LANES

    mesh = plsc.VectorSubcoreMesh(core_axis_name="c", subcore_axis_name="s")

    @functools.partial(
        pl.kernel,
        mesh=mesh,
        out_type=jax.ShapeDtypeStruct((n_fine, f), jnp.float32),
        scratch_types=[
            pltpu.VMEM((per_w * ips,), jnp.int32),
            pltpu.VMEM((2, ips, f), jnp.float32),
            pltpu.VMEM((2, _S, f), jnp.float32),
            pltpu.VMEM((f,), jnp.float32),
            pltpu.SemaphoreType.DMA,
            pltpu.SemaphoreType.DMA,
        ],
    )
    def sc_kernel(table_hbm, idxf_hbm, b_hbm, out_hbm, idx_v, gbuf, obuf,
                  bias_v, gsem, osem):
        wid = lax.axis_index("s") * nc + lax.axis_index("c")
        start = wid * per_w
        n_t = jnp.minimum(per_w, n_sc_total - start)
        pltpu.sync_copy(idxf_hbm.at[pl.ds(start * ips, per_w * ips)], idx_v)
        pltpu.sync_copy(b_hbm, bias_v)
        bias_regs = [bias_v[pl.ds(_LANES * j, _LANES)] for j in range(jpf)]

        def issue_gather(t, buf):
            pltpu.async_copy(table_hbm.at[idx_v.at[pl.ds(t * ips, ips)]],
                             gbuf.at[buf], gsem)

        issue_gather(0, 0)
        issue_gather(1, 1)

        def slot(t, buf):
            @pl.when(t < n_t)
            def _():
                pltpu.make_async_copy(table_hbm.at[pl.ds(0, ips)],
                                      gbuf.at[buf], gsem).wait()

                @pl.when(t >= 2)
                def _wait_wb():
                    pltpu.make_async_copy(obuf.at[buf],
                                          out_hbm.at[pl.ds(0, _S)],
                                          osem).wait()

                def row(i, carry):
                    for j in range(jpf):
                        sl = pl.ds(_LANES * j, _LANES)
                        acc = gbuf[buf, k_nbr * i, sl]
                        for k in range(1, k_nbr):
                            acc = acc + gbuf[buf, k_nbr * i + k, sl]
                        obuf[buf, i, sl] = jnp.maximum(
                            acc + bias_regs[j], 0.0)
                    return carry

                lax.fori_loop(0, _S, row, 0)

                pltpu.async_copy(obuf.at[buf],
                                 out_hbm.at[pl.ds((start + t) * _S, _S)],
                                 osem)

                @pl.when(t + 2 < n_t)
                def _next():
                    issue_gather(t + 2, buf)

        def outer(t0, carry):
            slot(t0 * 2, 0)
            slot(t0 * 2 + 1, 1)
            return carry

        lax.fori_loop(0, per_w // 2, outer, 0)

        for _ in range(2):
            pltpu.make_async_copy(obuf.at[0], out_hbm.at[pl.ds(0, _S)],
                                  osem).wait()

    return sc_kernel


def kernel(lv_coarse, ls_coarse, ls_fine, W, b):
    n_coarse, d = lv_coarse.shape
    n_fine, k_nbr = ls_fine.shape
    f = W.shape[1]

    w_blocks = W.reshape(k_nbr, d, f)
    p = _project_tables(lv_coarse, w_blocks)
    table = p.reshape(k_nbr * n_coarse, f)

    info = plsc.get_sparse_core_info()
    nw = info.num_cores * info.num_subcores
    n_sc_total = n_fine // _S
    per_w = -(-n_sc_total // nw)
    per_w += per_w % 2

    idx = ls_fine.astype(jnp.int32)
    idx = idx + (jnp.arange(k_nbr, dtype=jnp.int32) * n_coarse)[None, :]
    idxf = idx.reshape(-1)
    pad = nw * per_w * _S * k_nbr - idxf.shape[0]
    idxf = jnp.pad(idxf, (0, pad))

    return _make_sc_gather_sum(k_nbr, f, n_fine)(table, idxf, b)


# speedup vs baseline: 3.4076x; 1.4466x over previous
import functools

import jax
import jax.numpy as jnp
from jax import lax
from jax.experimental import pallas as pl
from jax.experimental.pallas import tpu as pltpu
from jax.experimental.pallas import tpu_sc as plsc

_LANES = 16
_S = 32
_NBUF = 3


def _chunks_per_worker(n_fine, nw):
    per_w = -(-(n_fine // _S) // nw)
    return per_w + (-per_w) % _NBUF


def _project_tables(lv_coarse, w_blocks):
    n, d = lv_coarse.shape
    k_nbr, _, f = w_blocks.shape
    rb = 1000 if n % 1000 == 0 else 8
    assert n % rb == 0

    def body(lv_ref, w_ref, p_ref):
        p_ref[0] = jnp.dot(lv_ref[...], w_ref[0],
                           preferred_element_type=jnp.float32)

    return pl.pallas_call(
        body,
        grid=(n // rb, k_nbr),
        in_specs=[
            pl.BlockSpec((rb, d), lambda r, k: (r, 0)),
            pl.BlockSpec((1, d, f), lambda r, k: (k, 0, 0)),
        ],
        out_specs=pl.BlockSpec((1, rb, f), lambda r, k: (k, r, 0)),
        out_shape=jax.ShapeDtypeStruct((k_nbr, n, f), jnp.float32),
        compiler_params=pltpu.CompilerParams(
            dimension_semantics=("parallel", "arbitrary")),
    )(lv_coarse, w_blocks)


def _make_sc_gather_sum(k_nbr, f, n_fine):
    info = plsc.get_sparse_core_info()
    nc, ns = info.num_cores, info.num_subcores
    nw = nc * ns
    ips = _S * k_nbr
    n_sc_total = n_fine // _S
    assert n_fine % _S == 0
    nbuf = _NBUF
    per_w = _chunks_per_worker(n_fine, nw)
    jpf = f // _LANES

    mesh = plsc.VectorSubcoreMesh(core_axis_name="c", subcore_axis_name="s")

    @functools.partial(
        pl.kernel,
        mesh=mesh,
        out_type=jax.ShapeDtypeStruct((n_fine, f), jnp.float32),
        scratch_types=[
            pltpu.VMEM((per_w * ips,), jnp.int32),
            pltpu.VMEM((nbuf, ips, f), jnp.float32),
            pltpu.VMEM((nbuf, _S, f), jnp.float32),
            pltpu.VMEM((f,), jnp.float32),
            pltpu.SemaphoreType.DMA,
            pltpu.SemaphoreType.DMA,
        ],
    )
    def sc_kernel(table_hbm, idxf_hbm, b_hbm, out_hbm, idx_v, gbuf, obuf,
                  bias_v, gsem, osem):
        wid = lax.axis_index("s") * nc + lax.axis_index("c")
        start = wid * per_w
        n_t = jnp.minimum(per_w, n_sc_total - start)
        pltpu.sync_copy(idxf_hbm.at[pl.ds(start * ips, per_w * ips)], idx_v)
        pltpu.sync_copy(b_hbm, bias_v)
        bias_regs = [bias_v[pl.ds(_LANES * j, _LANES)] for j in range(jpf)]

        def issue_gather(t, buf):
            pltpu.async_copy(table_hbm.at[idx_v.at[pl.ds(t * ips, ips)]],
                             gbuf.at[buf], gsem)

        for i in range(nbuf - 1):
            issue_gather(i, i)

        def slot(t, buf):
            @pl.when(t < n_t)
            def _():
                @pl.when(t + nbuf - 1 < n_t)
                def _next():
                    issue_gather(t + nbuf - 1, (buf + nbuf - 1) % nbuf)

                pltpu.make_async_copy(table_hbm.at[pl.ds(0, ips)],
                                      gbuf.at[buf], gsem).wait()

                @pl.when(t >= nbuf)
                def _wait_wb():
                    pltpu.make_async_copy(obuf.at[buf],
                                          out_hbm.at[pl.ds(0, _S)],
                                          osem).wait()

                @plsc.parallel_loop(0, _S, unroll=4)
                def row(i):
                    for j in range(jpf):
                        sl = pl.ds(_LANES * j, _LANES)
                        acc = gbuf[buf, k_nbr * i, sl]
                        for k in range(1, k_nbr):
                            acc = acc + gbuf[buf, k_nbr * i + k, sl]
                        obuf[buf, i, sl] = jnp.maximum(
                            acc + bias_regs[j], 0.0)

                pltpu.async_copy(obuf.at[buf],
                                 out_hbm.at[pl.ds((start + t) * _S, _S)],
                                 osem)

        def outer(t0, carry):
            for b in range(nbuf):
                slot(t0 * nbuf + b, b)
            return carry

        lax.fori_loop(0, per_w // nbuf, outer, 0)

        for _ in range(nbuf):
            pltpu.make_async_copy(obuf.at[0], out_hbm.at[pl.ds(0, _S)],
                                  osem).wait()

    return sc_kernel


def kernel(lv_coarse, ls_coarse, ls_fine, W, b):
    n_coarse, d = lv_coarse.shape
    n_fine, k_nbr = ls_fine.shape
    f = W.shape[1]

    w_blocks = W.reshape(k_nbr, d, f)
    p = _project_tables(lv_coarse, w_blocks)
    table = p.reshape(k_nbr * n_coarse, f)

    info = plsc.get_sparse_core_info()
    nw = info.num_cores * info.num_subcores
    per_w = _chunks_per_worker(n_fine, nw)

    idx = ls_fine.astype(jnp.int32)
    idx = idx + (jnp.arange(k_nbr, dtype=jnp.int32) * n_coarse)[None, :]
    idxf = idx.reshape(-1)
    pad = nw * per_w * _S * k_nbr - idxf.shape[0]
    idxf = jnp.pad(idxf, (0, pad))

    return _make_sc_gather_sum(k_nbr, f, n_fine)(table, idxf, b)


# speedup vs baseline: 3.4091x; 1.0004x over previous
import functools

import jax
import jax.numpy as jnp
from jax import lax
from jax.experimental import pallas as pl
from jax.experimental.pallas import tpu as pltpu
from jax.experimental.pallas import tpu_sc as plsc

_LANES = 16
_S = 32
_NBUF = 3


def _chunks_per_worker(n_fine, nw):
    per_w = -(-(n_fine // _S) // nw)
    return per_w + (-per_w) % _NBUF


def _project_tables(lv_coarse, w_blocks):
    n, d = lv_coarse.shape
    k_nbr, _, f = w_blocks.shape
    rb = 1000 if n % 1000 == 0 else 8
    assert n % rb == 0
    nrb = n // rb

    def body(lv_ref, w_ref, p_ref):
        p_ref[...] = jnp.dot(lv_ref[...], w_ref[0],
                             preferred_element_type=jnp.float32)

    return pl.pallas_call(
        body,
        grid=(nrb, k_nbr),
        in_specs=[
            pl.BlockSpec((rb, d), lambda r, k: (r, 0)),
            pl.BlockSpec((1, d, f), lambda r, k: (k, 0, 0)),
        ],
        out_specs=pl.BlockSpec((rb, f), lambda r, k: (k * nrb + r, 0)),
        out_shape=jax.ShapeDtypeStruct((k_nbr * n, f), jnp.float32),
        compiler_params=pltpu.CompilerParams(
            dimension_semantics=("parallel", "arbitrary")),
    )(lv_coarse, w_blocks)


def _make_sc_gather_sum(k_nbr, f, n_fine):
    info = plsc.get_sparse_core_info()
    nc, ns = info.num_cores, info.num_subcores
    nw = nc * ns
    ips = _S * k_nbr
    n_sc_total = n_fine // _S
    assert n_fine % _S == 0
    nbuf = _NBUF
    per_w = _chunks_per_worker(n_fine, nw)
    jpf = f // _LANES

    mesh = plsc.VectorSubcoreMesh(core_axis_name="c", subcore_axis_name="s")

    @functools.partial(
        pl.kernel,
        mesh=mesh,
        out_type=jax.ShapeDtypeStruct((n_fine, f), jnp.float32),
        scratch_types=[
            pltpu.VMEM((per_w * ips,), jnp.int32),
            pltpu.VMEM((nbuf, ips, f), jnp.float32),
            pltpu.VMEM((nbuf, _S, f), jnp.float32),
            pltpu.VMEM((f,), jnp.float32),
            pltpu.SemaphoreType.DMA,
            pltpu.SemaphoreType.DMA,
        ],
    )
    def sc_kernel(table_hbm, idxf_hbm, b_hbm, out_hbm, idx_v, gbuf, obuf,
                  bias_v, gsem, osem):
        wid = lax.axis_index("s") * nc + lax.axis_index("c")
        start = wid * per_w
        n_t = jnp.minimum(per_w, n_sc_total - start)
        pltpu.sync_copy(idxf_hbm.at[pl.ds(start * ips, per_w * ips)], idx_v)
        pltpu.sync_copy(b_hbm, bias_v)
        bias_regs = [bias_v[pl.ds(_LANES * j, _LANES)] for j in range(jpf)]

        def issue_gather(t, buf):
            pltpu.async_copy(table_hbm.at[idx_v.at[pl.ds(t * ips, ips)]],
                             gbuf.at[buf], gsem)

        for i in range(nbuf - 1):
            issue_gather(i, i)

        def slot(t, buf):
            @pl.when(t < n_t)
            def _():
                @pl.when(t + nbuf - 1 < n_t)
                def _next():
                    issue_gather(t + nbuf - 1, (buf + nbuf - 1) % nbuf)

                pltpu.make_async_copy(table_hbm.at[pl.ds(0, ips)],
                                      gbuf.at[buf], gsem).wait()

                @pl.when(t >= nbuf)
                def _wait_wb():
                    pltpu.make_async_copy(obuf.at[buf],
                                          out_hbm.at[pl.ds(0, _S)],
                                          osem).wait()

                @plsc.parallel_loop(0, _S, unroll=4)
                def row(i):
                    for j in range(jpf):
                        sl = pl.ds(_LANES * j, _LANES)
                        acc = gbuf[buf, k_nbr * i, sl]
                        for k in range(1, k_nbr):
                            acc = acc + gbuf[buf, k_nbr * i + k, sl]
                        obuf[buf, i, sl] = jnp.maximum(
                            acc + bias_regs[j], 0.0)

                pltpu.async_copy(obuf.at[buf],
                                 out_hbm.at[pl.ds((start + t) * _S, _S)],
                                 osem)

        def outer(t0, carry):
            for b in range(nbuf):
                slot(t0 * nbuf + b, b)
            return carry

        lax.fori_loop(0, per_w // nbuf, outer, 0)

        for _ in range(nbuf):
            pltpu.make_async_copy(obuf.at[0], out_hbm.at[pl.ds(0, _S)],
                                  osem).wait()

    return sc_kernel


def kernel(lv_coarse, ls_coarse, ls_fine, W, b):
    n_coarse, d = lv_coarse.shape
    n_fine, k_nbr = ls_fine.shape
    f = W.shape[1]

    w_blocks = W.reshape(k_nbr, d, f)
    table = _project_tables(lv_coarse, w_blocks)

    info = plsc.get_sparse_core_info()
    nw = info.num_cores * info.num_subcores
    per_w = _chunks_per_worker(n_fine, nw)

    idx = ls_fine.astype(jnp.int32)
    idx = idx + (jnp.arange(k_nbr, dtype=jnp.int32) * n_coarse)[None, :]
    idxf = idx.reshape(-1)
    pad = nw * per_w * _S * k_nbr - idxf.shape[0]
    idxf = jnp.pad(idxf, (0, pad))

    return _make_sc_gather_sum(k_nbr, f, n_fine)(table, idxf, b)


# speedup vs baseline: 3.9610x; 1.1619x over previous
import functools

import jax
import jax.numpy as jnp
from jax import lax
from jax.experimental import pallas as pl
from jax.experimental.pallas import tpu as pltpu
from jax.experimental.pallas import tpu_sc as plsc

_LANES = 16
_S = 32
_NBUF = 3


def _chunks_per_worker(n_fine, nw):
    per_w = -(-(n_fine // _S) // nw)
    return per_w + (-per_w) % _NBUF


def _project_tables(lv_coarse, w_blocks):
    n, d = lv_coarse.shape
    k_nbr, _, f = w_blocks.shape
    rb = 400 if n % 400 == 0 else 8
    assert n % rb == 0
    nrb = n // rb

    def body(lv_ref, w_ref, p_ref):
        lv = lv_ref[...]
        for k in range(k_nbr):
            p_ref[k] = jnp.dot(lv, w_ref[k],
                               preferred_element_type=jnp.float32)

    p = pl.pallas_call(
        body,
        grid=(nrb,),
        in_specs=[
            pl.BlockSpec((rb, d), lambda r: (r, 0)),
            pl.BlockSpec((k_nbr, d, f), lambda r: (0, 0, 0)),
        ],
        out_specs=pl.BlockSpec((k_nbr, rb, f), lambda r: (0, r, 0)),
        out_shape=jax.ShapeDtypeStruct((k_nbr, n, f), jnp.float32),
        compiler_params=pltpu.CompilerParams(
            dimension_semantics=("arbitrary",)),
    )(lv_coarse, w_blocks)
    return p.reshape(k_nbr * n, f)


def _make_sc_gather_sum(k_nbr, f, n_fine):
    info = plsc.get_sparse_core_info()
    nc, ns = info.num_cores, info.num_subcores
    nw = nc * ns
    ips = _S * k_nbr
    n_sc_total = n_fine // _S
    assert n_fine % _S == 0
    nbuf = _NBUF
    per_w = _chunks_per_worker(n_fine, nw)
    jpf = f // _LANES

    mesh = plsc.VectorSubcoreMesh(core_axis_name="c", subcore_axis_name="s")

    @functools.partial(
        pl.kernel,
        mesh=mesh,
        out_type=jax.ShapeDtypeStruct((n_fine, f), jnp.float32),
        scratch_types=[
            pltpu.VMEM((per_w * ips,), jnp.int32),
            pltpu.VMEM((nbuf, ips, f), jnp.float32),
            pltpu.VMEM((nbuf, _S, f), jnp.float32),
            pltpu.VMEM((f,), jnp.float32),
            pltpu.SemaphoreType.DMA,
            pltpu.SemaphoreType.DMA,
        ],
    )
    def sc_kernel(table_hbm, idxf_hbm, b_hbm, out_hbm, idx_v, gbuf, obuf,
                  bias_v, gsem, osem):
        wid = lax.axis_index("s") * nc + lax.axis_index("c")
        start = wid * per_w
        n_t = jnp.minimum(per_w, n_sc_total - start)
        pltpu.sync_copy(idxf_hbm.at[pl.ds(start * ips, per_w * ips)], idx_v)
        pltpu.sync_copy(b_hbm, bias_v)
        bias_regs = [bias_v[pl.ds(_LANES * j, _LANES)] for j in range(jpf)]

        def issue_gather(t, buf):
            pltpu.async_copy(table_hbm.at[idx_v.at[pl.ds(t * ips, ips)]],
                             gbuf.at[buf], gsem)

        for i in range(nbuf - 1):
            issue_gather(i, i)

        def slot(t, buf):
            @pl.when(t < n_t)
            def _():
                @pl.when(t + nbuf - 1 < n_t)
                def _next():
                    issue_gather(t + nbuf - 1, (buf + nbuf - 1) % nbuf)

                pltpu.make_async_copy(table_hbm.at[pl.ds(0, ips)],
                                      gbuf.at[buf], gsem).wait()

                @pl.when(t >= nbuf)
                def _wait_wb():
                    pltpu.make_async_copy(obuf.at[buf],
                                          out_hbm.at[pl.ds(0, _S)],
                                          osem).wait()

                @plsc.parallel_loop(0, _S, unroll=4)
                def row(i):
                    for j in range(jpf):
                        sl = pl.ds(_LANES * j, _LANES)
                        acc = gbuf[buf, k_nbr * i, sl]
                        for k in range(1, k_nbr):
                            acc = acc + gbuf[buf, k_nbr * i + k, sl]
                        obuf[buf, i, sl] = jnp.maximum(
                            acc + bias_regs[j], 0.0)

                pltpu.async_copy(obuf.at[buf],
                                 out_hbm.at[pl.ds((start + t) * _S, _S)],
                                 osem)

        def outer(t0, carry):
            for b in range(nbuf):
                slot(t0 * nbuf + b, b)
            return carry

        lax.fori_loop(0, per_w // nbuf, outer, 0)

        for _ in range(nbuf):
            pltpu.make_async_copy(obuf.at[0], out_hbm.at[pl.ds(0, _S)],
                                  osem).wait()

    return sc_kernel


def kernel(lv_coarse, ls_coarse, ls_fine, W, b):
    n_coarse, d = lv_coarse.shape
    n_fine, k_nbr = ls_fine.shape
    f = W.shape[1]

    w_blocks = W.reshape(k_nbr, d, f)
    table = _project_tables(lv_coarse, w_blocks)

    info = plsc.get_sparse_core_info()
    nw = info.num_cores * info.num_subcores
    per_w = _chunks_per_worker(n_fine, nw)

    idx = ls_fine.astype(jnp.int32)
    idx = idx + (jnp.arange(k_nbr, dtype=jnp.int32) * n_coarse)[None, :]
    idxf = idx.reshape(-1)
    pad = nw * per_w * _S * k_nbr - idxf.shape[0]
    idxf = jnp.pad(idxf, (0, pad))

    return _make_sc_gather_sum(k_nbr, f, n_fine)(table, idxf, b)


# speedup vs baseline: 3.9677x; 1.0017x over previous
import functools

import jax
import jax.numpy as jnp
from jax import lax
from jax.experimental import pallas as pl
from jax.experimental.pallas import tpu as pltpu
from jax.experimental.pallas import tpu_sc as plsc

_LANES = 16
_S = 32
_NBUF = 3


def _chunks_per_worker(n_fine, nw):
    per_w = -(-(n_fine // _S) // nw)
    return per_w + (-per_w) % _NBUF


def _project_tables(lv_coarse, w_blocks):
    n, d = lv_coarse.shape
    k_nbr, _, f = w_blocks.shape
    rb = 400 if n % 400 == 0 else 8
    assert n % rb == 0
    nrb = n // rb

    def body(lv_ref, w_ref, p_ref):
        lv = lv_ref[...]
        for k in range(k_nbr):
            p_ref[k] = jnp.dot(lv, w_ref[k],
                               preferred_element_type=jnp.float32)

    p = pl.pallas_call(
        body,
        grid=(nrb,),
        in_specs=[
            pl.BlockSpec((rb, d), lambda r: (r, 0)),
            pl.BlockSpec((k_nbr, d, f), lambda r: (0, 0, 0)),
        ],
        out_specs=pl.BlockSpec((k_nbr, rb, f), lambda r: (0, r, 0)),
        out_shape=jax.ShapeDtypeStruct((k_nbr, n, f), jnp.float32),
        compiler_params=pltpu.CompilerParams(
            dimension_semantics=("arbitrary",)),
    )(lv_coarse, w_blocks)
    return p.reshape(k_nbr * n, f)


def _make_sc_gather_sum(k_nbr, f, n_fine):
    info = plsc.get_sparse_core_info()
    nc, ns = info.num_cores, info.num_subcores
    nw = nc * ns
    ips = _S * k_nbr
    n_sc_total = n_fine // _S
    assert n_fine % _S == 0
    nbuf = _NBUF
    per_w = _chunks_per_worker(n_fine, nw)
    jpf = f // _LANES

    mesh = plsc.VectorSubcoreMesh(core_axis_name="c", subcore_axis_name="s")

    @functools.partial(
        pl.kernel,
        mesh=mesh,
        out_type=jax.ShapeDtypeStruct((n_fine, f), jnp.float32),
        scratch_types=[
            pltpu.VMEM((per_w * ips,), jnp.int32),
            pltpu.VMEM((nbuf, ips, f), jnp.float32),
            pltpu.VMEM((nbuf, _S, f), jnp.float32),
            pltpu.VMEM((f,), jnp.float32),
            pltpu.SemaphoreType.DMA,
            pltpu.SemaphoreType.DMA,
        ],
    )
    def sc_kernel(table_hbm, idxf_hbm, b_hbm, out_hbm, idx_v, gbuf, obuf,
                  bias_v, gsem, osem):
        wid = lax.axis_index("s") * nc + lax.axis_index("c")
        start = wid * per_w
        n_t = jnp.minimum(per_w, n_sc_total - start)
        pltpu.sync_copy(idxf_hbm.at[pl.ds(start * ips, per_w * ips)], idx_v)
        pltpu.sync_copy(b_hbm, bias_v)
        bias_regs = [bias_v[pl.ds(_LANES * j, _LANES)] for j in range(jpf)]

        def issue_gather(t, buf):
            pltpu.async_copy(table_hbm.at[idx_v.at[pl.ds(t * ips, ips)]],
                             gbuf.at[buf], gsem)

        for i in range(nbuf - 1):
            issue_gather(i, i)

        def slot(t, buf):
            @pl.when(t < n_t)
            def _():
                @pl.when(t + nbuf - 1 < n_t)
                def _next():
                    issue_gather(t + nbuf - 1, (buf + nbuf - 1) % nbuf)

                pltpu.make_async_copy(table_hbm.at[pl.ds(0, ips)],
                                      gbuf.at[buf], gsem).wait()

                @pl.when(t >= nbuf)
                def _wait_wb():
                    pltpu.make_async_copy(obuf.at[buf],
                                          out_hbm.at[pl.ds(0, _S)],
                                          osem).wait()

                @plsc.parallel_loop(0, _S, unroll=4)
                def row(i):
                    for j in range(jpf):
                        sl = pl.ds(_LANES * j, _LANES)
                        acc = gbuf[buf, k_nbr * i, sl]
                        for k in range(1, k_nbr):
                            acc = acc + gbuf[buf, k_nbr * i + k, sl]
                        obuf[buf, i, sl] = jnp.maximum(
                            acc + bias_regs[j], 0.0)

                pltpu.async_copy(obuf.at[buf],
                                 out_hbm.at[pl.ds((start + t) * _S, _S)],
                                 osem)

        def outer(t0, carry):
            for b in range(nbuf):
                slot(t0 * nbuf + b, b)
            return carry

        lax.fori_loop(0, per_w // nbuf, outer, 0)

        for _ in range(nbuf):
            pltpu.make_async_copy(obuf.at[0], out_hbm.at[pl.ds(0, _S)],
                                  osem).wait()

    return sc_kernel


def kernel(lv_coarse, ls_coarse, ls_fine, W, b):
    n_coarse, d = lv_coarse.shape
    n_fine, k_nbr = ls_fine.shape
    f = W.shape[1]

    w_blocks = W.reshape(k_nbr, d, f)
    table = _project_tables(lv_coarse, w_blocks)

    info = plsc.get_sparse_core_info()
    nw = info.num_cores * info.num_subcores
    per_w = _chunks_per_worker(n_fine, nw)

    idxf = ls_fine.astype(jnp.int32).reshape(-1)
    n_idx = idxf.shape[0]
    offs = jnp.remainder(jnp.arange(n_idx, dtype=jnp.int32),
                         jnp.int32(k_nbr)) * jnp.int32(n_coarse)
    idxf = idxf + offs
    pad = nw * per_w * _S * k_nbr - n_idx
    idxf = jnp.pad(idxf, (0, pad))

    return _make_sc_gather_sum(k_nbr, f, n_fine)(table, idxf, b)


# speedup vs baseline: 5.1712x; 1.3033x over previous
import functools

import jax
import jax.numpy as jnp
from jax import lax
from jax.experimental import pallas as pl
from jax.experimental.pallas import tpu as pltpu
from jax.experimental.pallas import tpu_sc as plsc

_LANES = 16
_S = 32
_NBUF = 3


def _chunks_per_worker(n_fine, nw):
    per_w = -(-(n_fine // _S) // nw)
    return per_w + (-per_w) % _NBUF


def _project_tables(lv_coarse, w_blocks):
    n, d = lv_coarse.shape
    k_nbr, _, f = w_blocks.shape
    rb = 400 if n % 400 == 0 else 8
    assert n % rb == 0
    nrb = n // rb

    def body(lv_ref, w_ref, p_ref):
        lv = lv_ref[...]
        for k in range(k_nbr):
            p_ref[k] = jnp.dot(lv, w_ref[k],
                               preferred_element_type=jnp.float32)

    p = pl.pallas_call(
        body,
        grid=(nrb,),
        in_specs=[
            pl.BlockSpec((rb, d), lambda r: (r, 0)),
            pl.BlockSpec((k_nbr, d, f), lambda r: (0, 0, 0)),
        ],
        out_specs=pl.BlockSpec((k_nbr, rb, f), lambda r: (0, r, 0)),
        out_shape=jax.ShapeDtypeStruct((k_nbr, n, f), jnp.float32),
        compiler_params=pltpu.CompilerParams(
            dimension_semantics=("arbitrary",)),
    )(lv_coarse, w_blocks)
    return p.reshape(k_nbr * n, f)


def _make_sc_gather_sum(k_nbr, f, n_fine):
    info = plsc.get_sparse_core_info()
    nc, ns = info.num_cores, info.num_subcores
    nw = nc * ns
    ips = _S * k_nbr
    n_sc_total = n_fine // _S
    assert n_fine % _S == 0
    nbuf = _NBUF
    per_w = _chunks_per_worker(n_fine, nw)
    jpf = f // _LANES

    mesh = plsc.VectorSubcoreMesh(core_axis_name="c", subcore_axis_name="s")

    @functools.partial(
        pl.kernel,
        mesh=mesh,
        out_type=jax.ShapeDtypeStruct((n_fine, f), jnp.float32),
        scratch_types=[
            pltpu.VMEM((per_w * ips,), jnp.int32),
            pltpu.VMEM((nbuf, ips, f), jnp.float32),
            pltpu.VMEM((nbuf, _S, f), jnp.float32),
            pltpu.VMEM((f,), jnp.float32),
            pltpu.SemaphoreType.DMA,
            pltpu.SemaphoreType.DMA,
        ],
    )
    def sc_kernel(table_hbm, idxf_hbm, b_hbm, out_hbm, idx_v, gbuf, obuf,
                  bias_v, gsem, osem):
        wid = lax.axis_index("s") * nc + lax.axis_index("c")
        start = wid * per_w
        n_t = jnp.minimum(per_w, n_sc_total - start)
        ncolpad = nw * per_w * _S
        ipw = per_w * _S
        for k in range(k_nbr):
            pltpu.async_copy(
                idxf_hbm.at[pl.ds(k * ncolpad + start * _S, ipw)],
                idx_v.at[pl.ds(k * ipw, ipw)], gsem)
        pltpu.sync_copy(b_hbm, bias_v)
        pltpu.make_async_copy(idxf_hbm.at[pl.ds(0, k_nbr * ipw)], idx_v,
                              gsem).wait()
        bias_regs = [bias_v[pl.ds(_LANES * j, _LANES)] for j in range(jpf)]

        def issue_gather(t, buf):
            for k in range(k_nbr):
                pltpu.async_copy(
                    table_hbm.at[idx_v.at[pl.ds(k * ipw + t * _S, _S)]],
                    gbuf.at[buf, pl.ds(k * _S, _S)], gsem)

        for i in range(nbuf - 1):
            issue_gather(i, i)

        def slot(t, buf):
            @pl.when(t < n_t)
            def _():
                @pl.when(t + nbuf - 1 < n_t)
                def _next():
                    issue_gather(t + nbuf - 1, (buf + nbuf - 1) % nbuf)

                pltpu.make_async_copy(table_hbm.at[pl.ds(0, ips)],
                                      gbuf.at[buf], gsem).wait()

                @pl.when(t >= nbuf)
                def _wait_wb():
                    pltpu.make_async_copy(obuf.at[buf],
                                          out_hbm.at[pl.ds(0, _S)],
                                          osem).wait()

                @plsc.parallel_loop(0, _S, unroll=4)
                def row(i):
                    for j in range(jpf):
                        sl = pl.ds(_LANES * j, _LANES)
                        acc = gbuf[buf, i, sl]
                        for k in range(1, k_nbr):
                            acc = acc + gbuf[buf, k * _S + i, sl]
                        obuf[buf, i, sl] = jnp.maximum(
                            acc + bias_regs[j], 0.0)

                pltpu.async_copy(obuf.at[buf],
                                 out_hbm.at[pl.ds((start + t) * _S, _S)],
                                 osem)

        def outer(t0, carry):
            for b in range(nbuf):
                slot(t0 * nbuf + b, b)
            return carry

        lax.fori_loop(0, per_w // nbuf, outer, 0)

        for _ in range(nbuf):
            pltpu.make_async_copy(obuf.at[0], out_hbm.at[pl.ds(0, _S)],
                                  osem).wait()

    return sc_kernel


def kernel(lv_coarse, ls_coarse, ls_fine, W, b):
    n_coarse, d = lv_coarse.shape
    n_fine, k_nbr = ls_fine.shape
    f = W.shape[1]

    w_blocks = W.reshape(k_nbr, d, f)
    table = _project_tables(lv_coarse, w_blocks)

    info = plsc.get_sparse_core_info()
    nw = info.num_cores * info.num_subcores
    per_w = _chunks_per_worker(n_fine, nw)

    ncolpad = nw * per_w * _S
    cols = jnp.pad(ls_fine.T.astype(jnp.int32),
                   ((0, 0), (0, ncolpad - n_fine)))
    idxf = cols.reshape(-1)
    offs = (jnp.arange(k_nbr * ncolpad, dtype=jnp.int32)
            // jnp.int32(ncolpad)) * jnp.int32(n_coarse)
    idxf = idxf + offs

    return _make_sc_gather_sum(k_nbr, f, n_fine)(table, idxf, b)


# speedup vs baseline: 6.2332x; 1.2054x over previous
import functools

import jax
import jax.numpy as jnp
from jax import lax
from jax.experimental import pallas as pl
from jax.experimental.pallas import tpu as pltpu
from jax.experimental.pallas import tpu_sc as plsc

_LANES = 16
_S = 32
_NBUF = 3


def _chunks_per_worker(n_fine, nw):
    per_w = -(-(n_fine // _S) // nw)
    return per_w + (-per_w) % _NBUF


def _project_tables(lv_coarse, w_blocks):
    n, d = lv_coarse.shape
    k_nbr, _, f = w_blocks.shape
    rb = 1000 if n % 1000 == 0 else 8
    assert n % rb == 0
    nrb = n // rb

    def body(lv_ref, w_ref, p_ref):
        lv = lv_ref[...]
        for k in range(k_nbr):
            p_ref[k] = jnp.dot(lv, w_ref[k],
                               preferred_element_type=jnp.float32)

    p = pl.pallas_call(
        body,
        grid=(nrb,),
        in_specs=[
            pl.BlockSpec((rb, d), lambda r: (r, 0)),
            pl.BlockSpec((k_nbr, d, f), lambda r: (0, 0, 0)),
        ],
        out_specs=pl.BlockSpec((k_nbr, rb, f), lambda r: (0, r, 0)),
        out_shape=jax.ShapeDtypeStruct((k_nbr, n, f), jnp.float32),
        compiler_params=pltpu.CompilerParams(
            dimension_semantics=("arbitrary",)),
    )(lv_coarse, w_blocks)
    return p.reshape(k_nbr * n, f)


def _make_sc_gather_sum(k_nbr, f, n_fine):
    info = plsc.get_sparse_core_info()
    nc, ns = info.num_cores, info.num_subcores
    nw = nc * ns
    ips = _S * k_nbr
    n_sc_total = n_fine // _S
    assert n_fine % _S == 0
    nbuf = _NBUF
    per_w = _chunks_per_worker(n_fine, nw)
    jpf = f // _LANES

    mesh = plsc.VectorSubcoreMesh(core_axis_name="c", subcore_axis_name="s")

    @functools.partial(
        pl.kernel,
        mesh=mesh,
        out_type=jax.ShapeDtypeStruct((n_fine, f), jnp.float32),
        scratch_types=[
            pltpu.VMEM((per_w * ips,), jnp.int32),
            pltpu.VMEM((nbuf, ips, f), jnp.float32),
            pltpu.VMEM((nbuf, _S, f), jnp.float32),
            pltpu.VMEM((f,), jnp.float32),
            pltpu.SemaphoreType.DMA,
            pltpu.SemaphoreType.DMA,
        ],
    )
    def sc_kernel(table_hbm, idxf_hbm, b_hbm, out_hbm, idx_v, gbuf, obuf,
                  bias_v, gsem, osem):
        wid = lax.axis_index("s") * nc + lax.axis_index("c")
        start = wid * per_w
        n_t = jnp.minimum(per_w, n_sc_total - start)
        ncolpad = nw * per_w * _S
        ipw = per_w * _S
        for k in range(k_nbr):
            pltpu.async_copy(
                idxf_hbm.at[pl.ds(k * ncolpad + start * _S, ipw)],
                idx_v.at[pl.ds(k * ipw, ipw)], gsem)
        pltpu.sync_copy(b_hbm, bias_v)
        pltpu.make_async_copy(idxf_hbm.at[pl.ds(0, k_nbr * ipw)], idx_v,
                              gsem).wait()
        bias_regs = [bias_v[pl.ds(_LANES * j, _LANES)] for j in range(jpf)]

        def issue_gather(t, buf):
            for k in range(k_nbr):
                pltpu.async_copy(
                    table_hbm.at[idx_v.at[pl.ds(k * ipw + t * _S, _S)]],
                    gbuf.at[buf, pl.ds(k * _S, _S)], gsem)

        for i in range(nbuf - 1):
            issue_gather(i, i)

        def slot(t, buf):
            @pl.when(t < n_t)
            def _():
                @pl.when(t + nbuf - 1 < n_t)
                def _next():
                    issue_gather(t + nbuf - 1, (buf + nbuf - 1) % nbuf)

                pltpu.make_async_copy(table_hbm.at[pl.ds(0, ips)],
                                      gbuf.at[buf], gsem).wait()

                @pl.when(t >= nbuf)
                def _wait_wb():
                    pltpu.make_async_copy(obuf.at[buf],
                                          out_hbm.at[pl.ds(0, _S)],
                                          osem).wait()

                @plsc.parallel_loop(0, _S, unroll=4)
                def row(i):
                    for j in range(jpf):
                        sl = pl.ds(_LANES * j, _LANES)
                        acc = gbuf[buf, i, sl]
                        for k in range(1, k_nbr):
                            acc = acc + gbuf[buf, k * _S + i, sl]
                        obuf[buf, i, sl] = jnp.maximum(
                            acc + bias_regs[j], 0.0)

                pltpu.async_copy(obuf.at[buf],
                                 out_hbm.at[pl.ds((start + t) * _S, _S)],
                                 osem)

        def outer(t0, carry):
            for b in range(nbuf):
                slot(t0 * nbuf + b, b)
            return carry

        lax.fori_loop(0, per_w // nbuf, outer, 0)

        for _ in range(nbuf):
            pltpu.make_async_copy(obuf.at[0], out_hbm.at[pl.ds(0, _S)],
                                  osem).wait()

    return sc_kernel


def kernel(lv_coarse, ls_coarse, ls_fine, W, b):
    n_coarse, d = lv_coarse.shape
    n_fine, k_nbr = ls_fine.shape
    f = W.shape[1]

    w_blocks = W.reshape(k_nbr, d, f)
    table = _project_tables(lv_coarse, w_blocks)

    info = plsc.get_sparse_core_info()
    nw = info.num_cores * info.num_subcores
    per_w = _chunks_per_worker(n_fine, nw)

    ncolpad = nw * per_w * _S
    cols = jnp.pad(ls_fine.T.astype(jnp.int32),
                   ((0, 0), (0, ncolpad - n_fine)))
    idxf = cols.reshape(-1)
    offs = (jnp.arange(k_nbr * ncolpad, dtype=jnp.int32)
            // jnp.int32(ncolpad)) * jnp.int32(n_coarse)
    idxf = idxf + offs

    return _make_sc_gather_sum(k_nbr, f, n_fine)(table, idxf, b)


# speedup vs baseline: 6.9808x; 1.1199x over previous
import functools

import jax
import jax.numpy as jnp
from jax import lax
from jax.experimental import pallas as pl
from jax.experimental.pallas import tpu as pltpu
from jax.experimental.pallas import tpu_sc as plsc

_LANES = 16
_S = 32
_NBUF = 3


def _chunks_per_worker(n_fine, nw):
    per_w = -(-(n_fine // _S) // nw)
    return per_w + (-per_w) % _NBUF


def _project_tables(lv_coarse, w_blocks):
    n, d = lv_coarse.shape
    k_nbr, _, f = w_blocks.shape
    rb = 2000 if n % 2000 == 0 else 8
    assert n % rb == 0
    nrb = n // rb

    def body(lv_ref, w_ref, p_ref):
        lv = lv_ref[...]
        for k in range(k_nbr):
            p_ref[k] = jnp.dot(lv, w_ref[k],
                               preferred_element_type=jnp.float32)

    p = pl.pallas_call(
        body,
        grid=(nrb,),
        in_specs=[
            pl.BlockSpec((rb, d), lambda r: (r, 0)),
            pl.BlockSpec((k_nbr, d, f), lambda r: (0, 0, 0)),
        ],
        out_specs=pl.BlockSpec((k_nbr, rb, f), lambda r: (0, r, 0)),
        out_shape=jax.ShapeDtypeStruct((k_nbr, n, f), jnp.float32),
        compiler_params=pltpu.CompilerParams(
            dimension_semantics=("arbitrary",)),
    )(lv_coarse, w_blocks)
    return p.reshape(k_nbr * n, f)


def _make_sc_gather_sum(k_nbr, f, n_fine):
    info = plsc.get_sparse_core_info()
    nc, ns = info.num_cores, info.num_subcores
    nw = nc * ns
    ips = _S * k_nbr
    n_sc_total = n_fine // _S
    assert n_fine % _S == 0
    nbuf = _NBUF
    per_w = _chunks_per_worker(n_fine, nw)
    jpf = f // _LANES

    mesh = plsc.VectorSubcoreMesh(core_axis_name="c", subcore_axis_name="s")

    @functools.partial(
        pl.kernel,
        mesh=mesh,
        out_type=jax.ShapeDtypeStruct((n_fine, f), jnp.float32),
        scratch_types=[
            pltpu.VMEM((per_w * ips,), jnp.int32),
            pltpu.VMEM((nbuf, ips, f), jnp.float32),
            pltpu.VMEM((nbuf, _S, f), jnp.float32),
            pltpu.VMEM((f,), jnp.float32),
            pltpu.SemaphoreType.DMA,
            pltpu.SemaphoreType.DMA,
        ],
    )
    def sc_kernel(table_hbm, idxf_hbm, b_hbm, out_hbm, idx_v, gbuf, obuf,
                  bias_v, gsem, osem):
        wid = lax.axis_index("s") * nc + lax.axis_index("c")
        start = wid * per_w
        n_t = jnp.minimum(per_w, n_sc_total - start)
        ncolpad = nw * per_w * _S
        ipw = per_w * _S
        for k in range(k_nbr):
            pltpu.async_copy(
                idxf_hbm.at[pl.ds(k * ncolpad + start * _S, ipw)],
                idx_v.at[pl.ds(k * ipw, ipw)], gsem)
        pltpu.sync_copy(b_hbm, bias_v)
        pltpu.make_async_copy(idxf_hbm.at[pl.ds(0, k_nbr * ipw)], idx_v,
                              gsem).wait()
        bias_regs = [bias_v[pl.ds(_LANES * j, _LANES)] for j in range(jpf)]

        def issue_gather(t, buf):
            for k in range(k_nbr):
                pltpu.async_copy(
                    table_hbm.at[idx_v.at[pl.ds(k * ipw + t * _S, _S)]],
                    gbuf.at[buf, pl.ds(k * _S, _S)], gsem)

        for i in range(nbuf - 1):
            issue_gather(i, i)

        def slot(t, buf):
            @pl.when(t < n_t)
            def _():
                @pl.when(t + nbuf - 1 < n_t)
                def _next():
                    issue_gather(t + nbuf - 1, (buf + nbuf - 1) % nbuf)

                pltpu.make_async_copy(table_hbm.at[pl.ds(0, ips)],
                                      gbuf.at[buf], gsem).wait()

                @pl.when(t >= nbuf)
                def _wait_wb():
                    pltpu.make_async_copy(obuf.at[buf],
                                          out_hbm.at[pl.ds(0, _S)],
                                          osem).wait()

                @plsc.parallel_loop(0, _S, unroll=4)
                def row(i):
                    for j in range(jpf):
                        sl = pl.ds(_LANES * j, _LANES)
                        acc = gbuf[buf, i, sl]
                        for k in range(1, k_nbr):
                            acc = acc + gbuf[buf, k * _S + i, sl]
                        obuf[buf, i, sl] = jnp.maximum(
                            acc + bias_regs[j], 0.0)

                pltpu.async_copy(obuf.at[buf],
                                 out_hbm.at[pl.ds((start + t) * _S, _S)],
                                 osem)

        def outer(t0, carry):
            for b in range(nbuf):
                slot(t0 * nbuf + b, b)
            return carry

        lax.fori_loop(0, per_w // nbuf, outer, 0)

        for _ in range(nbuf):
            pltpu.make_async_copy(obuf.at[0], out_hbm.at[pl.ds(0, _S)],
                                  osem).wait()

    return sc_kernel


def kernel(lv_coarse, ls_coarse, ls_fine, W, b):
    n_coarse, d = lv_coarse.shape
    n_fine, k_nbr = ls_fine.shape
    f = W.shape[1]

    w_blocks = W.reshape(k_nbr, d, f)
    table = _project_tables(lv_coarse, w_blocks)

    info = plsc.get_sparse_core_info()
    nw = info.num_cores * info.num_subcores
    per_w = _chunks_per_worker(n_fine, nw)

    ncolpad = nw * per_w * _S
    cols = jnp.pad(ls_fine.T.astype(jnp.int32),
                   ((0, 0), (0, ncolpad - n_fine)))
    idxf = cols.reshape(-1)
    offs = (jnp.arange(k_nbr * ncolpad, dtype=jnp.int32)
            // jnp.int32(ncolpad)) * jnp.int32(n_coarse)
    idxf = idxf + offs

    return _make_sc_gather_sum(k_nbr, f, n_fine)(table, idxf, b)


# speedup vs baseline: 7.2180x; 1.0340x over previous
import functools

import jax
import jax.numpy as jnp
from jax import lax
from jax.experimental import pallas as pl
from jax.experimental.pallas import tpu as pltpu
from jax.experimental.pallas import tpu_sc as plsc

_LANES = 16
_S = 32
_NBUF = 3


def _chunks_per_worker(n_fine, nw):
    per_w = -(-(n_fine // _S) // nw)
    return per_w + (-per_w) % _NBUF


def _project_tables(lv_coarse, w_blocks):
    n, d = lv_coarse.shape
    k_nbr, _, f = w_blocks.shape
    rb = 10000 if n % 10000 == 0 else 8
    assert n % rb == 0
    nrb = n // rb

    def body(lv_ref, w_ref, p_ref):
        lv = lv_ref[...]
        for k in range(k_nbr):
            p_ref[k] = jnp.dot(lv, w_ref[k],
                               preferred_element_type=jnp.float32)

    p = pl.pallas_call(
        body,
        grid=(nrb,),
        in_specs=[
            pl.BlockSpec((rb, d), lambda r: (r, 0)),
            pl.BlockSpec((k_nbr, d, f), lambda r: (0, 0, 0)),
        ],
        out_specs=pl.BlockSpec((k_nbr, rb, f), lambda r: (0, r, 0)),
        out_shape=jax.ShapeDtypeStruct((k_nbr, n, f), jnp.float32),
        compiler_params=pltpu.CompilerParams(
            dimension_semantics=("arbitrary",)),
    )(lv_coarse, w_blocks)
    return p.reshape(k_nbr * n, f)


def _make_sc_gather_sum(k_nbr, f, n_fine):
    info = plsc.get_sparse_core_info()
    nc, ns = info.num_cores, info.num_subcores
    nw = nc * ns
    ips = _S * k_nbr
    n_sc_total = n_fine // _S
    assert n_fine % _S == 0
    nbuf = _NBUF
    per_w = _chunks_per_worker(n_fine, nw)
    jpf = f // _LANES

    mesh = plsc.VectorSubcoreMesh(core_axis_name="c", subcore_axis_name="s")

    @functools.partial(
        pl.kernel,
        mesh=mesh,
        out_type=jax.ShapeDtypeStruct((n_fine, f), jnp.float32),
        scratch_types=[
            pltpu.VMEM((per_w * ips,), jnp.int32),
            pltpu.VMEM((nbuf, ips, f), jnp.float32),
            pltpu.VMEM((nbuf, _S, f), jnp.float32),
            pltpu.VMEM((f,), jnp.float32),
            pltpu.SemaphoreType.DMA,
            pltpu.SemaphoreType.DMA,
        ],
    )
    def sc_kernel(table_hbm, idxf_hbm, b_hbm, out_hbm, idx_v, gbuf, obuf,
                  bias_v, gsem, osem):
        wid = lax.axis_index("s") * nc + lax.axis_index("c")
        start = wid * per_w
        n_t = jnp.minimum(per_w, n_sc_total - start)
        ncolpad = nw * per_w * _S
        ipw = per_w * _S
        for k in range(k_nbr):
            pltpu.async_copy(
                idxf_hbm.at[pl.ds(k * ncolpad + start * _S, ipw)],
                idx_v.at[pl.ds(k * ipw, ipw)], gsem)
        pltpu.sync_copy(b_hbm, bias_v)
        pltpu.make_async_copy(idxf_hbm.at[pl.ds(0, k_nbr * ipw)], idx_v,
                              gsem).wait()
        bias_regs = [bias_v[pl.ds(_LANES * j, _LANES)] for j in range(jpf)]

        def issue_gather(t, buf):
            for k in range(k_nbr):
                pltpu.async_copy(
                    table_hbm.at[idx_v.at[pl.ds(k * ipw + t * _S, _S)]],
                    gbuf.at[buf, pl.ds(k * _S, _S)], gsem)

        for i in range(nbuf - 1):
            issue_gather(i, i)

        def slot(t, buf):
            @pl.when(t < n_t)
            def _():
                @pl.when(t + nbuf - 1 < n_t)
                def _next():
                    issue_gather(t + nbuf - 1, (buf + nbuf - 1) % nbuf)

                pltpu.make_async_copy(table_hbm.at[pl.ds(0, ips)],
                                      gbuf.at[buf], gsem).wait()

                @pl.when(t >= nbuf)
                def _wait_wb():
                    pltpu.make_async_copy(obuf.at[buf],
                                          out_hbm.at[pl.ds(0, _S)],
                                          osem).wait()

                @plsc.parallel_loop(0, _S, unroll=4)
                def row(i):
                    for j in range(jpf):
                        sl = pl.ds(_LANES * j, _LANES)
                        acc = gbuf[buf, i, sl]
                        for k in range(1, k_nbr):
                            acc = acc + gbuf[buf, k * _S + i, sl]
                        obuf[buf, i, sl] = jnp.maximum(
                            acc + bias_regs[j], 0.0)

                pltpu.async_copy(obuf.at[buf],
                                 out_hbm.at[pl.ds((start + t) * _S, _S)],
                                 osem)

        def outer(t0, carry):
            for b in range(nbuf):
                slot(t0 * nbuf + b, b)
            return carry

        lax.fori_loop(0, per_w // nbuf, outer, 0)

        for _ in range(nbuf):
            pltpu.make_async_copy(obuf.at[0], out_hbm.at[pl.ds(0, _S)],
                                  osem).wait()

    return sc_kernel


def kernel(lv_coarse, ls_coarse, ls_fine, W, b):
    n_coarse, d = lv_coarse.shape
    n_fine, k_nbr = ls_fine.shape
    f = W.shape[1]

    w_blocks = W.reshape(k_nbr, d, f)
    table = _project_tables(lv_coarse, w_blocks)

    info = plsc.get_sparse_core_info()
    nw = info.num_cores * info.num_subcores
    per_w = _chunks_per_worker(n_fine, nw)

    ncolpad = nw * per_w * _S
    cols = jnp.pad(ls_fine.T.astype(jnp.int32),
                   ((0, 0), (0, ncolpad - n_fine)))
    idxf = cols.reshape(-1)
    offs = (jnp.arange(k_nbr * ncolpad, dtype=jnp.int32)
            // jnp.int32(ncolpad)) * jnp.int32(n_coarse)
    idxf = idxf + offs

    return _make_sc_gather_sum(k_nbr, f, n_fine)(table, idxf, b)


# speedup vs baseline: 7.2665x; 1.0067x over previous
import functools

import jax
import jax.numpy as jnp
from jax import lax
from jax.experimental import pallas as pl
from jax.experimental.pallas import tpu as pltpu
from jax.experimental.pallas import tpu_sc as plsc

_LANES = 16
_S = 32
_NBUF = 4


def _chunks_per_worker(n_fine, nw):
    per_w = -(-(n_fine // _S) // nw)
    return per_w + (-per_w) % _NBUF


def _project_tables(lv_coarse, w_blocks):
    n, d = lv_coarse.shape
    k_nbr, _, f = w_blocks.shape
    rb = 10000 if n % 10000 == 0 else 8
    assert n % rb == 0
    nrb = n // rb

    def body(lv_ref, w_ref, p_ref):
        lv = lv_ref[...]
        for k in range(k_nbr):
            p_ref[k] = jnp.dot(lv, w_ref[k],
                               preferred_element_type=jnp.float32)

    p = pl.pallas_call(
        body,
        grid=(nrb,),
        in_specs=[
            pl.BlockSpec((rb, d), lambda r: (r, 0)),
            pl.BlockSpec((k_nbr, d, f), lambda r: (0, 0, 0)),
        ],
        out_specs=pl.BlockSpec((k_nbr, rb, f), lambda r: (0, r, 0)),
        out_shape=jax.ShapeDtypeStruct((k_nbr, n, f), jnp.float32),
        compiler_params=pltpu.CompilerParams(
            dimension_semantics=("arbitrary",)),
    )(lv_coarse, w_blocks)
    return p.reshape(k_nbr * n, f)


def _make_sc_gather_sum(k_nbr, f, n_fine):
    info = plsc.get_sparse_core_info()
    nc, ns = info.num_cores, info.num_subcores
    nw = nc * ns
    ips = _S * k_nbr
    n_sc_total = n_fine // _S
    assert n_fine % _S == 0
    nbuf = _NBUF
    per_w = _chunks_per_worker(n_fine, nw)
    jpf = f // _LANES

    mesh = plsc.VectorSubcoreMesh(core_axis_name="c", subcore_axis_name="s")

    @functools.partial(
        pl.kernel,
        mesh=mesh,
        out_type=jax.ShapeDtypeStruct((n_fine, f), jnp.float32),
        scratch_types=[
            pltpu.VMEM((per_w * ips,), jnp.int32),
            pltpu.VMEM((nbuf, ips, f), jnp.float32),
            pltpu.VMEM((nbuf, _S, f), jnp.float32),
            pltpu.VMEM((f,), jnp.float32),
            pltpu.SemaphoreType.DMA,
            pltpu.SemaphoreType.DMA,
        ],
    )
    def sc_kernel(table_hbm, idxf_hbm, b_hbm, out_hbm, idx_v, gbuf, obuf,
                  bias_v, gsem, osem):
        wid = lax.axis_index("s") * nc + lax.axis_index("c")
        start = wid * per_w
        n_t = jnp.minimum(per_w, n_sc_total - start)
        ncolpad = nw * per_w * _S
        ipw = per_w * _S
        for k in range(k_nbr):
            pltpu.async_copy(
                idxf_hbm.at[pl.ds(k * ncolpad + start * _S, ipw)],
                idx_v.at[pl.ds(k * ipw, ipw)], gsem)
        pltpu.sync_copy(b_hbm, bias_v)
        pltpu.make_async_copy(idxf_hbm.at[pl.ds(0, k_nbr * ipw)], idx_v,
                              gsem).wait()
        bias_regs = [bias_v[pl.ds(_LANES * j, _LANES)] for j in range(jpf)]

        def issue_gather(t, buf):
            for k in range(k_nbr):
                pltpu.async_copy(
                    table_hbm.at[idx_v.at[pl.ds(k * ipw + t * _S, _S)]],
                    gbuf.at[buf, pl.ds(k * _S, _S)], gsem)

        for i in range(nbuf - 1):
            issue_gather(i, i)

        def slot(t, buf):
            @pl.when(t < n_t)
            def _():
                @pl.when(t + nbuf - 1 < n_t)
                def _next():
                    issue_gather(t + nbuf - 1, (buf + nbuf - 1) % nbuf)

                pltpu.make_async_copy(table_hbm.at[pl.ds(0, ips)],
                                      gbuf.at[buf], gsem).wait()

                @pl.when(t >= nbuf)
                def _wait_wb():
                    pltpu.make_async_copy(obuf.at[buf],
                                          out_hbm.at[pl.ds(0, _S)],
                                          osem).wait()

                @plsc.parallel_loop(0, _S, unroll=4)
                def row(i):
                    for j in range(jpf):
                        sl = pl.ds(_LANES * j, _LANES)
                        acc = gbuf[buf, i, sl]
                        for k in range(1, k_nbr):
                            acc = acc + gbuf[buf, k * _S + i, sl]
                        obuf[buf, i, sl] = jnp.maximum(
                            acc + bias_regs[j], 0.0)

                pltpu.async_copy(obuf.at[buf],
                                 out_hbm.at[pl.ds((start + t) * _S, _S)],
                                 osem)

        def outer(t0, carry):
            for b in range(nbuf):
                slot(t0 * nbuf + b, b)
            return carry

        lax.fori_loop(0, per_w // nbuf, outer, 0)

        for _ in range(nbuf):
            pltpu.make_async_copy(obuf.at[0], out_hbm.at[pl.ds(0, _S)],
                                  osem).wait()

    return sc_kernel


def kernel(lv_coarse, ls_coarse, ls_fine, W, b):
    n_coarse, d = lv_coarse.shape
    n_fine, k_nbr = ls_fine.shape
    f = W.shape[1]

    w_blocks = W.reshape(k_nbr, d, f)
    table = _project_tables(lv_coarse, w_blocks)

    info = plsc.get_sparse_core_info()
    nw = info.num_cores * info.num_subcores
    per_w = _chunks_per_worker(n_fine, nw)

    ncolpad = nw * per_w * _S
    cols = jnp.pad(ls_fine.T.astype(jnp.int32),
                   ((0, 0), (0, ncolpad - n_fine)))
    idxf = cols.reshape(-1)
    offs = (jnp.arange(k_nbr * ncolpad, dtype=jnp.int32)
            // jnp.int32(ncolpad)) * jnp.int32(n_coarse)
    idxf = idxf + offs

    return _make_sc_gather_sum(k_nbr, f, n_fine)(table, idxf, b)
